# baseline (speedup 1.0000x reference)
import functools

import jax
import jax.numpy as jnp
from jax.experimental import pallas as pl
from jax.experimental.pallas import tpu as pltpu

_SELU_SCALE = 1.0507009873554805
_SELU_ALPHA = 1.6732632423543772
_QLEVELS = 254.0


def _pass1_body(nI, n, x_ref, adj_ref, W1_ref, b1_ref, W2_ref,
                adjq_ref, s2_ref, corr_ref, s1_s, csum_s):
    t = pl.program_id(0)
    R = adj_ref.shape[0]

    @pl.when(t == 0)
    def _():
        s1_s[...] = jnp.dot(x_ref[...], W1_ref[...],
                            preferred_element_type=jnp.float32
                            ).astype(jnp.bfloat16)
        csum_s[...] = jnp.zeros_like(csum_s)

    a = adj_ref[...]
    adjq_ref[...] = (a - 0.5).astype(jnp.float8_e4m3fn)
    acc = jnp.dot(a.astype(jnp.bfloat16), s1_s[...],
                  preferred_element_type=jnp.float32)
    h = jnp.maximum(acc + b1_ref[...], 0.0)
    v = jnp.dot(h, W2_ref[...], preferred_element_type=jnp.float32)
    s2_ref[...] = v.astype(jnp.float8_e4m3fn)
    row = jax.lax.broadcasted_iota(jnp.int32, (R, 1), 0) + t * R
    vm = jnp.where(row < n, v, 0.0)
    csum_s[...] = csum_s[...] + jnp.sum(vm, axis=0, keepdims=True)

    @pl.when(t == nI - 1)
    def _():
        corr_ref[...] = csum_s[...]


def _pass2_body(nI, R, n, nheads,
                adjq_ref, s2_ref, corr_ref, b2_ref, sub_ref, Wfc_ref, bfc_ref,
                Watt_ref, batt_ref, aatt_ref, out_ref, sum_s):
    t = pl.program_id(0)

    acc = jnp.dot(adjq_ref[...], s2_ref[...],
                  preferred_element_type=jnp.float32)
    pre = acc + (0.5 * corr_ref[...] + b2_ref[...])
    h2 = jnp.maximum(pre, 0.0)
    row = jax.lax.broadcasted_iota(jnp.int32, (R, 1), 0) + t * R
    h2 = jnp.where(row < n, h2, 0.0)
    psum = jnp.sum(h2, axis=0, keepdims=True)

    @pl.when(t == 0)
    def _():
        sum_s[...] = psum

    @pl.when(t > 0)
    def _():
        sum_s[...] = sum_s[...] + psum

    @pl.when(t == nI - 1)
    def _head():
        v = sum_s[...] * (1.0 / n)
        g = _SELU_SCALE * jnp.where(v > 0.0, v,
                                    _SELU_ALPHA * (jnp.exp(v) - 1.0))
        x_ext = jnp.dot(sub_ref[...], Wfc_ref[...],
                        preferred_element_type=jnp.float32) + bfc_ref[...]
        z = jnp.concatenate([g, x_ext], axis=1)
        heads = jnp.concatenate(
            [jnp.dot(z, Watt_ref[h], preferred_element_type=jnp.float32)
             + batt_ref[h:h + 1, :]
             for h in range(nheads)], axis=0)
        e = jnp.sum(jnp.tanh(heads) * aatt_ref[...], axis=1,
                    keepdims=True)
        m = jnp.max(e, axis=0, keepdims=True)
        ex = jnp.exp(e - m)
        alpha = ex / jnp.sum(ex, axis=0, keepdims=True)
        fused = jnp.sum(alpha * heads, axis=0, keepdims=True)
        mo = jnp.max(fused, axis=1, keepdims=True)
        lse = jnp.log(jnp.sum(jnp.exp(fused - mo), axis=1, keepdims=True)) + mo
        out_ref[...] = fused - lse


def kernel(x, adj, sub_fea, W1, b1, W2, b2, Wfc, bfc, Watt, batt, a_att):
    n, nfeat = x.shape
    nhid = W1.shape[1]
    nheads, _, nclass = Watt.shape
    R = 256
    nI = pl.cdiv(n, R)

    b1r = b1.reshape(1, -1)
    b2r = b2.reshape(1, -1)
    bfcr = bfc.reshape(1, -1)

    const = lambda shape: pl.BlockSpec(shape, lambda t: tuple(0 for _ in shape))

    body1 = functools.partial(_pass1_body, nI, n)
    adj_q, s2, corr = pl.pallas_call(
        body1,
        grid=(nI,),
        in_specs=[
            const((n, nfeat)),
            pl.BlockSpec((R, n), lambda t: (t, 0)),
            const(W1.shape),
            const(b1r.shape),
            const(W2.shape),
        ],
        out_specs=[
            pl.BlockSpec((R, n), lambda t: (t, 0)),
            pl.BlockSpec((R, 2 * nhid), lambda t: (t, 0)),
            pl.BlockSpec((1, 2 * nhid), lambda t: (0, 0)),
        ],
        out_shape=[
            jax.ShapeDtypeStruct((n, n), jnp.float8_e4m3fn),
            jax.ShapeDtypeStruct((n, 2 * nhid), jnp.float8_e4m3fn),
            jax.ShapeDtypeStruct((1, 2 * nhid), jnp.float32),
        ],
        scratch_shapes=[
            pltpu.VMEM((n, nhid), jnp.bfloat16),
            pltpu.VMEM((1, 2 * nhid), jnp.float32),
        ],
    )(x, adj, W1, b1r, W2)

    R2 = 512
    nI2 = pl.cdiv(n, R2)
    body2 = functools.partial(_pass2_body, nI2, R2, n, nheads)
    return pl.pallas_call(
        body2,
        grid=(nI2,),
        in_specs=[
            pl.BlockSpec((R2, n), lambda t: (t, 0)),
            const((n, 2 * nhid)),
            const((1, 2 * nhid)),
            const(b2r.shape),
            const(sub_fea.shape),
            const(Wfc.shape),
            const(bfcr.shape),
            const(Watt.shape),
            const(batt.shape),
            const(a_att.shape),
        ],
        out_specs=pl.BlockSpec((1, nclass), lambda t: (0, 0)),
        out_shape=jax.ShapeDtypeStruct((1, nclass), jnp.float32),
        scratch_shapes=[
            pltpu.VMEM((1, 2 * nhid), jnp.float32),
        ],
    )(adj_q, s2, corr, b2r, sub_fea, Wfc, bfcr, Watt, batt, a_att)


# speedup vs baseline: 1.0618x; 1.0618x over previous
import functools

import jax
import jax.numpy as jnp
from jax.experimental import pallas as pl
from jax.experimental.pallas import tpu as pltpu

_SELU_SCALE = 1.0507009873554805
_SELU_ALPHA = 1.6732632423543772
_QLEVELS = 254.0


def _pass1_body(nI, n, x_ref, adj_ref, W1_ref, b1_ref, W2_ref,
                adjq_ref, s2_ref, corr_ref, s1_s, csum_s):
    t = pl.program_id(0)
    R = adj_ref.shape[0]

    @pl.when(t == 0)
    def _():
        s1_s[...] = jnp.dot(x_ref[...], W1_ref[...],
                            preferred_element_type=jnp.float32
                            ).astype(jnp.bfloat16)
        csum_s[...] = jnp.zeros_like(csum_s)

    a = adj_ref[...]
    adjq_ref[...] = (a - 0.5).astype(jnp.float8_e4m3fn)
    acc = jnp.dot(a.astype(jnp.bfloat16), s1_s[...],
                  preferred_element_type=jnp.float32)
    h = jnp.maximum(acc + b1_ref[...], 0.0)
    v = jnp.dot(h, W2_ref[...], preferred_element_type=jnp.float32)
    s2_ref[...] = v.astype(jnp.float8_e4m3fn)
    row = jax.lax.broadcasted_iota(jnp.int32, (R, 1), 0) + t * R
    vm = jnp.where(row < n, v, 0.0)
    csum_s[...] = csum_s[...] + jnp.sum(vm, axis=0, keepdims=True)

    @pl.when(t == nI - 1)
    def _():
        corr_ref[...] = csum_s[...]


def _pass2_body(nI, R, n, nheads,
                adjq_ref, s2_ref, corr_ref, b2_ref, sub_ref, Wfc_ref, bfc_ref,
                Watt_ref, batt_ref, aatt_ref, out_ref, sum_s):
    t = pl.program_id(0)

    acc = jnp.dot(adjq_ref[...], s2_ref[...],
                  preferred_element_type=jnp.float32)
    pre = acc + (0.5 * corr_ref[...] + b2_ref[...])
    h2 = jnp.maximum(pre, 0.0)
    row = jax.lax.broadcasted_iota(jnp.int32, (R, 1), 0) + t * R
    h2 = jnp.where(row < n, h2, 0.0)
    psum = jnp.sum(h2, axis=0, keepdims=True)

    @pl.when(t == 0)
    def _():
        sum_s[...] = psum

    @pl.when(t > 0)
    def _():
        sum_s[...] = sum_s[...] + psum

    @pl.when(t == nI - 1)
    def _head():
        v = sum_s[...] * (1.0 / n)
        g = _SELU_SCALE * jnp.where(v > 0.0, v,
                                    _SELU_ALPHA * (jnp.exp(v) - 1.0))
        x_ext = jnp.dot(sub_ref[...], Wfc_ref[...],
                        preferred_element_type=jnp.float32) + bfc_ref[...]
        z = jnp.concatenate([g, x_ext], axis=1)
        heads = jnp.concatenate(
            [jnp.dot(z, Watt_ref[h], preferred_element_type=jnp.float32)
             + batt_ref[h:h + 1, :]
             for h in range(nheads)], axis=0)
        e = jnp.sum(jnp.tanh(heads) * aatt_ref[...], axis=1,
                    keepdims=True)
        m = jnp.max(e, axis=0, keepdims=True)
        ex = jnp.exp(e - m)
        alpha = ex / jnp.sum(ex, axis=0, keepdims=True)
        fused = jnp.sum(alpha * heads, axis=0, keepdims=True)
        mo = jnp.max(fused, axis=1, keepdims=True)
        lse = jnp.log(jnp.sum(jnp.exp(fused - mo), axis=1, keepdims=True)) + mo
        out_ref[...] = fused - lse


def kernel(x, adj, sub_fea, W1, b1, W2, b2, Wfc, bfc, Watt, batt, a_att):
    n, nfeat = x.shape
    nhid = W1.shape[1]
    nheads, _, nclass = Watt.shape
    R = 512
    nI = pl.cdiv(n, R)

    b1r = b1.reshape(1, -1)
    b2r = b2.reshape(1, -1)
    bfcr = bfc.reshape(1, -1)

    const = lambda shape: pl.BlockSpec(shape, lambda t: tuple(0 for _ in shape))

    body1 = functools.partial(_pass1_body, nI, n)
    adj_q, s2, corr = pl.pallas_call(
        body1,
        grid=(nI,),
        in_specs=[
            const((n, nfeat)),
            pl.BlockSpec((R, n), lambda t: (t, 0)),
            const(W1.shape),
            const(b1r.shape),
            const(W2.shape),
        ],
        out_specs=[
            pl.BlockSpec((R, n), lambda t: (t, 0)),
            pl.BlockSpec((R, 2 * nhid), lambda t: (t, 0)),
            pl.BlockSpec((1, 2 * nhid), lambda t: (0, 0)),
        ],
        out_shape=[
            jax.ShapeDtypeStruct((n, n), jnp.float8_e4m3fn),
            jax.ShapeDtypeStruct((n, 2 * nhid), jnp.float8_e4m3fn),
            jax.ShapeDtypeStruct((1, 2 * nhid), jnp.float32),
        ],
        scratch_shapes=[
            pltpu.VMEM((n, nhid), jnp.bfloat16),
            pltpu.VMEM((1, 2 * nhid), jnp.float32),
        ],
        compiler_params=pltpu.CompilerParams(
            vmem_limit_bytes=110 * 1024 * 1024),
    )(x, adj, W1, b1r, W2)

    R2 = 1024
    nI2 = pl.cdiv(n, R2)
    body2 = functools.partial(_pass2_body, nI2, R2, n, nheads)
    return pl.pallas_call(
        body2,
        grid=(nI2,),
        in_specs=[
            pl.BlockSpec((R2, n), lambda t: (t, 0)),
            const((n, 2 * nhid)),
            const((1, 2 * nhid)),
            const(b2r.shape),
            const(sub_fea.shape),
            const(Wfc.shape),
            const(bfcr.shape),
            const(Watt.shape),
            const(batt.shape),
            const(a_att.shape),
        ],
        out_specs=pl.BlockSpec((1, nclass), lambda t: (0, 0)),
        out_shape=jax.ShapeDtypeStruct((1, nclass), jnp.float32),
        scratch_shapes=[
            pltpu.VMEM((1, 2 * nhid), jnp.float32),
        ],
    )(adj_q, s2, corr, b2r, sub_fea, Wfc, bfcr, Watt, batt, a_att)


# speedup vs baseline: 1.0626x; 1.0008x over previous
import functools

import jax
import jax.numpy as jnp
from jax.experimental import pallas as pl
from jax.experimental.pallas import tpu as pltpu

_SELU_SCALE = 1.0507009873554805
_SELU_ALPHA = 1.6732632423543772


def _pass1_body(nI, n, x_ref, adj_ref, W1_ref, b1_ref, W2_ref,
                adjq_ref, s2_ref, corr_ref, s1_s, csum_s):
    t = pl.program_id(0)
    R = adj_ref.shape[0]

    @pl.when(t == 0)
    def _():
        s1_s[...] = jnp.dot(x_ref[...], W1_ref[...],
                            preferred_element_type=jnp.float32
                            ).astype(jnp.bfloat16)
        csum_s[...] = jnp.zeros_like(csum_s)

    a = adj_ref[...]
    adjq_ref[...] = (a - 0.5).astype(jnp.float8_e4m3fn)
    acc = jnp.dot(a.astype(jnp.bfloat16), s1_s[...],
                  preferred_element_type=jnp.float32)
    h = jnp.maximum(acc + b1_ref[...], 0.0)
    v = jnp.dot(h, W2_ref[...], preferred_element_type=jnp.float32)
    s2_ref[...] = v.astype(jnp.float8_e4m3fn)
    row = jax.lax.broadcasted_iota(jnp.int32, (R, 1), 0) + t * R
    vm = jnp.where(row < n, v, 0.0)
    csum_s[...] = csum_s[...] + jnp.sum(vm, axis=0, keepdims=True)

    @pl.when(t == nI - 1)
    def _():
        corr_ref[...] = csum_s[...]


def _pass2_body(nI, R, n, nheads,
                adjq_ref, s2_ref, corr_ref, b2_ref, sub_ref, Wfc_ref, bfc_ref,
                Watt_ref, batt_ref, aatt_ref, out_ref, sum_s):
    t = pl.program_id(0)

    acc = jnp.dot(adjq_ref[...], s2_ref[...],
                  preferred_element_type=jnp.float32)
    pre = acc + (0.5 * corr_ref[...] + b2_ref[...])
    h2 = jnp.maximum(pre, 0.0)
    row = jax.lax.broadcasted_iota(jnp.int32, (R, 1), 0) + t * R
    h2 = jnp.where(row < n, h2, 0.0)
    psum = jnp.sum(h2, axis=0, keepdims=True)

    @pl.when(t == 0)
    def _():
        sum_s[...] = psum

    @pl.when(t > 0)
    def _():
        sum_s[...] = sum_s[...] + psum

    @pl.when(t == nI - 1)
    def _head():
        v = sum_s[...] * (1.0 / n)
        g = _SELU_SCALE * jnp.where(v > 0.0, v,
                                    _SELU_ALPHA * (jnp.exp(v) - 1.0))
        x_ext = jnp.dot(sub_ref[...], Wfc_ref[...],
                        preferred_element_type=jnp.float32) + bfc_ref[...]
        z = jnp.concatenate([g, x_ext], axis=1)
        heads = jnp.concatenate(
            [jnp.dot(z, Watt_ref[h], preferred_element_type=jnp.float32)
             + batt_ref[h:h + 1, :]
             for h in range(nheads)], axis=0)
        e = jnp.sum(jnp.tanh(heads) * aatt_ref[...], axis=1,
                    keepdims=True)
        m = jnp.max(e, axis=0, keepdims=True)
        ex = jnp.exp(e - m)
        alpha = ex / jnp.sum(ex, axis=0, keepdims=True)
        fused = jnp.sum(alpha * heads, axis=0, keepdims=True)
        mo = jnp.max(fused, axis=1, keepdims=True)
        lse = jnp.log(jnp.sum(jnp.exp(fused - mo), axis=1, keepdims=True)) + mo
        out_ref[...] = fused - lse


def kernel(x, adj, sub_fea, W1, b1, W2, b2, Wfc, bfc, Watt, batt, a_att):
    n, nfeat = x.shape
    nhid = W1.shape[1]
    nheads, _, nclass = Watt.shape
    R = 512
    nI = pl.cdiv(n, R)

    b1r = b1.reshape(1, -1)
    b2r = b2.reshape(1, -1)
    bfcr = bfc.reshape(1, -1)

    const = lambda shape: pl.BlockSpec(shape, lambda t: tuple(0 for _ in shape))

    body1 = functools.partial(_pass1_body, nI, n)
    adj_q, s2, corr = pl.pallas_call(
        body1,
        grid=(nI,),
        in_specs=[
            const((n, nfeat)),
            pl.BlockSpec((R, n), lambda t: (t, 0)),
            const(W1.shape),
            const(b1r.shape),
            const(W2.shape),
        ],
        out_specs=[
            pl.BlockSpec((R, n), lambda t: (t, 0)),
            pl.BlockSpec((R, 2 * nhid), lambda t: (t, 0)),
            pl.BlockSpec((1, 2 * nhid), lambda t: (0, 0)),
        ],
        out_shape=[
            jax.ShapeDtypeStruct((n, n), jnp.float8_e4m3fn),
            jax.ShapeDtypeStruct((n, 2 * nhid), jnp.float8_e4m3fn),
            jax.ShapeDtypeStruct((1, 2 * nhid), jnp.float32),
        ],
        scratch_shapes=[
            pltpu.VMEM((n, nhid), jnp.bfloat16),
            pltpu.VMEM((1, 2 * nhid), jnp.float32),
        ],
        compiler_params=pltpu.CompilerParams(
            vmem_limit_bytes=64 * 1024 * 1024),
    )(x, adj, W1, b1r, W2)

    R2 = 1024
    nI2 = pl.cdiv(n, R2)
    body2 = functools.partial(_pass2_body, nI2, R2, n, nheads)
    return pl.pallas_call(
        body2,
        grid=(nI2,),
        in_specs=[
            pl.BlockSpec((R2, n), lambda t: (t, 0)),
            const((n, 2 * nhid)),
            const((1, 2 * nhid)),
            const(b2r.shape),
            const(sub_fea.shape),
            const(Wfc.shape),
            const(bfcr.shape),
            const(Watt.shape),
            const(batt.shape),
            const(a_att.shape),
        ],
        out_specs=pl.BlockSpec((1, nclass), lambda t: (0, 0)),
        out_shape=jax.ShapeDtypeStruct((1, nclass), jnp.float32),
        scratch_shapes=[
            pltpu.VMEM((1, 2 * nhid), jnp.float32),
        ],
    )(adj_q, s2, corr, b2r, sub_fea, Wfc, bfcr, Watt, batt, a_att)
